# Initial kernel scaffold; baseline (speedup 1.0000x reference)
import jax
import jax.numpy as jnp
from jax.experimental import pallas as pl


def kernel(de1, de2, f, de1_table, de2_table):
    raise NotImplementedError("write your pallas kernel here")


# speedup vs baseline: 1.9369x; 1.9369x over previous
import functools

import jax
import jax.numpy as jnp
from jax import lax
from jax.experimental import pallas as pl
from jax.experimental.pallas import tpu as pltpu
from jax.experimental.pallas import tpu_sc as plsc

NC, NS = 2, 16
NW = NC * NS
CHUNK = 128


@functools.partial(jax.jit, static_argnames=("n", "e"))
def _sc_lookup(de1, de2, f2d, t1, t2, *, n, e):
    per_w = n // NW
    n_chunks = per_w // CHUNK
    mesh = plsc.VectorSubcoreMesh(core_axis_name="c", subcore_axis_name="s")

    @functools.partial(
        pl.kernel,
        out_type=jax.ShapeDtypeStruct((n, 2 * e + 1), jnp.float32),
        mesh=mesh,
        scratch_types=[
            pltpu.VMEM((CHUNK,), jnp.int32),
            pltpu.VMEM((CHUNK,), jnp.int32),
            pltpu.VMEM((CHUNK, 1), jnp.float32),
            pltpu.VMEM((CHUNK, e), jnp.float32),
            pltpu.VMEM((CHUNK, e), jnp.float32),
            pltpu.SemaphoreType.DMA,
            pltpu.SemaphoreType.DMA,
        ],
        compiler_params=pltpu.CompilerParams(use_tc_tiling_on_sc=False),
    )
    def k(de1_hbm, de2_hbm, f_hbm, t1_hbm, t2_hbm, out_hbm,
          idx1_v, idx2_v, f_v, rows1_v, rows2_v, sem1, sem2):
        wid = lax.axis_index("s") * NC + lax.axis_index("c")
        base_w = wid * per_w

        def body(c, carry):
            base = base_w + c * CHUNK
            pltpu.sync_copy(de1_hbm.at[pl.ds(base, CHUNK)], idx1_v)
            pltpu.sync_copy(de2_hbm.at[pl.ds(base, CHUNK)], idx2_v)
            pltpu.sync_copy(f_hbm.at[pl.ds(base, CHUNK)], f_v)
            cp1 = pltpu.async_copy(t1_hbm.at[idx1_v], rows1_v, sem1)
            cp2 = pltpu.async_copy(t2_hbm.at[idx2_v], rows2_v, sem2)
            cp1.wait()
            cp2.wait()
            pltpu.sync_copy(rows1_v, out_hbm.at[pl.ds(base, CHUNK), pl.ds(0, e)])
            pltpu.sync_copy(rows2_v, out_hbm.at[pl.ds(base, CHUNK), pl.ds(e, e)])
            pltpu.sync_copy(f_v, out_hbm.at[pl.ds(base, CHUNK), pl.ds(2 * e, 1)])
            return carry

        lax.fori_loop(0, n_chunks, body, 0)

    return k(de1, de2, f2d, t1, t2)


def kernel(de1, de2, f, de1_table, de2_table):
    b, l = de1.shape
    _, e = de1_table.shape
    n = b * l
    out = _sc_lookup(
        de1.reshape(n), de2.reshape(n), f.reshape(n, 1),
        de1_table, de2_table, n=n, e=e)
    return out.reshape(b, l, 2 * e + 1)
